# baseline (speedup 1.0000x reference)
import functools

import jax
import jax.numpy as jnp
from jax import lax
from jax.experimental import pallas as pl
from jax.experimental.pallas import tpu as pltpu
from jax.experimental.pallas import tpu_sc as plsc

B = 4096
L = 200
D = 64
H = 128
O = 10
HALF = L // 2


def _pool_sc(text2, emb16):
    info = plsc.get_sparse_core_info()
    ncores = info.num_cores
    nw = ncores * info.num_subcores
    rpw = B // nw
    mesh = plsc.VectorSubcoreMesh(core_axis_name="c", subcore_axis_name="s")

    @functools.partial(
        pl.kernel,
        out_type=(jax.ShapeDtypeStruct((B, D // 2), jnp.float32),
                  jax.ShapeDtypeStruct((B, D // 2), jnp.float32)),
        mesh=mesh,
        compiler_params=pltpu.CompilerParams(use_tc_tiling_on_sc=False,
                                             needs_layout_passes=False),
        scratch_types=[
            pltpu.VMEM((2 * rpw, HALF), jnp.int32),
            pltpu.VMEM((L, D), jnp.bfloat16),
            pltpu.VMEM((L, D), jnp.bfloat16),
            pltpu.VMEM((rpw, D // 2), jnp.float32),
            pltpu.VMEM((rpw, D // 2), jnp.float32),
            pltpu.SemaphoreType.DMA,
            pltpu.SemaphoreType.DMA,
        ],
    )
    def pool(text_hbm, emb_hbm, oute_hbm, outo_hbm,
             idx_v, rows0, rows1, oute_v, outo_v, sem0, sem1):
        wid = lax.axis_index("s") * ncores + lax.axis_index("c")
        base = wid * rpw
        pltpu.sync_copy(text_hbm.at[pl.ds(2 * base, 2 * rpw)], idx_v)
        bufs = (rows0, rows1)
        sems = (sem0, sem1)

        def issue(b, t):
            pltpu.async_copy(emb_hbm.at[idx_v.at[2 * b]],
                             bufs[t].at[pl.ds(0, HALF)], sems[t])
            pltpu.async_copy(emb_hbm.at[idx_v.at[2 * b + 1]],
                             bufs[t].at[pl.ds(HALF, HALF)], sems[t])

        def drain(t):
            pltpu.make_async_copy(emb_hbm.at[pl.ds(0, L)], bufs[t], sems[t]).wait()

        def consume(b, t):
            drain(t)
            buf = bufs[t]
            zero = jnp.zeros((16,), jnp.float32)

            def rbody(r, acc):
                e0, o0, e1, o1 = acc
                lo = buf[r, pl.ds(0, 32)]
                hi = buf[r, pl.ds(32, 32)]
                le, lo_ = plsc.unpack(lo, format=plsc.PackFormat.INTERLEAVED)
                he, ho = plsc.unpack(hi, format=plsc.PackFormat.INTERLEAVED)
                return (e0 + le, o0 + lo_, e1 + he, o1 + ho)

            acc = lax.fori_loop(0, L, rbody, (zero,) * 4, unroll=8)
            inv = jnp.float32(1.0 / L)
            oute_v[b, pl.ds(0, 16)] = acc[0] * inv
            outo_v[b, pl.ds(0, 16)] = acc[1] * inv
            oute_v[b, pl.ds(16, 16)] = acc[2] * inv
            outo_v[b, pl.ds(16, 16)] = acc[3] * inv

        issue(0, 0)

        def outer(i, carry):
            for t in range(2):
                b = 2 * i + t

                @pl.when(b + 1 < rpw)
                def _():
                    issue(b + 1, (t + 1) % 2)

                consume(b, t)
            return carry

        lax.fori_loop(0, rpw // 2, outer, 0)
        pltpu.sync_copy(oute_v, oute_hbm.at[pl.ds(base, rpw)])
        pltpu.sync_copy(outo_v, outo_hbm.at[pl.ds(base, rpw)])

    return pool(text2, emb16)


def _mlp_body(xe_ref, xo_ref, w1e_ref, w1o_ref, b1_ref, w2_ref, b2_ref, o_ref):
    h = lax.dot_general(xe_ref[...], w1e_ref[...], (((1,), (1,)), ((), ())),
                        preferred_element_type=jnp.float32)
    h += lax.dot_general(xo_ref[...], w1o_ref[...], (((1,), (1,)), ((), ())),
                         preferred_element_type=jnp.float32)
    h = jnp.maximum(h + b1_ref[...], 0.0)
    o = lax.dot_general(h, w2_ref[...], (((1,), (1,)), ((), ())),
                        preferred_element_type=jnp.float32)
    o_ref[...] = o + b2_ref[...]


def _mlp_tc(pe, po, W1, b1, W2, b2):
    blk = 512
    Dh = D // 2
    return pl.pallas_call(
        _mlp_body,
        grid=(B // blk,),
        in_specs=[
            pl.BlockSpec((blk, Dh), lambda i: (i, 0)),
            pl.BlockSpec((blk, Dh), lambda i: (i, 0)),
            pl.BlockSpec((H, Dh), lambda i: (0, 0)),
            pl.BlockSpec((H, Dh), lambda i: (0, 0)),
            pl.BlockSpec((1, H), lambda i: (0, 0)),
            pl.BlockSpec((O, H), lambda i: (0, 0)),
            pl.BlockSpec((1, O), lambda i: (0, 0)),
        ],
        out_specs=pl.BlockSpec((blk, O), lambda i: (i, 0)),
        out_shape=jax.ShapeDtypeStruct((B, O), jnp.float32),
    )(pe, po, W1[:, 0::2], W1[:, 1::2], b1.reshape(1, H), W2, b2.reshape(1, O))


def kernel(text, emb, W1, b1, W2, b2):
    text2 = text.astype(jnp.int32).reshape(2 * B, HALF)
    emb16 = emb.astype(jnp.bfloat16)
    pe, po = _pool_sc(text2, emb16)
    return _mlp_tc(pe, po, W1, b1, W2, b2)


# speedup vs baseline: 1.1334x; 1.1334x over previous
import functools

import jax
import jax.numpy as jnp
from jax import lax
from jax.experimental import pallas as pl
from jax.experimental.pallas import tpu as pltpu
from jax.experimental.pallas import tpu_sc as plsc

B = 4096
L = 200
D = 64
H = 128
O = 10
HALF = L // 2
NVREG = D // 16
WPC = 4


def _pool_sc(gtext, pbits, emb2):
    info = plsc.get_sparse_core_info()
    ncores = info.num_cores
    nw = ncores * info.num_subcores
    rpw = B // nw
    nchunk = 2 * rpw
    mesh = plsc.VectorSubcoreMesh(core_axis_name="c", subcore_axis_name="s")

    @functools.partial(
        pl.kernel,
        out_type=jax.ShapeDtypeStruct((B, D), jnp.float32),
        mesh=mesh,
        scratch_types=[
            pltpu.VMEM((nchunk, HALF), jnp.int32),
            pltpu.VMEM((nchunk * WPC + 16,), jnp.int32),
            pltpu.VMEM((L, 2 * D), jnp.float32),
            pltpu.VMEM((L, 2 * D), jnp.float32),
            pltpu.VMEM((rpw, D), jnp.float32),
            pltpu.SemaphoreType.DMA,
            pltpu.SemaphoreType.DMA,
        ],
    )
    def pool(gtext_hbm, pbits_hbm, emb_hbm, out_hbm,
             gidx_v, bits_v, rows0, rows1, out_v, sem0, sem1):
        wid = lax.axis_index("s") * ncores + lax.axis_index("c")
        base = wid * rpw
        pltpu.sync_copy(gtext_hbm.at[pl.ds(2 * base, nchunk)], gidx_v)
        pltpu.sync_copy(pbits_hbm.at[pl.ds(2 * base * WPC, nchunk * WPC)],
                        bits_v.at[pl.ds(0, nchunk * WPC)])
        bufs = (rows0, rows1)
        sems = (sem0, sem1)

        def issue(b, t):
            pltpu.async_copy(emb_hbm.at[gidx_v.at[2 * b]],
                             bufs[t].at[pl.ds(0, HALF)], sems[t])
            pltpu.async_copy(emb_hbm.at[gidx_v.at[2 * b + 1]],
                             bufs[t].at[pl.ds(HALF, HALF)], sems[t])

        def drain(t):
            pltpu.make_async_copy(emb_hbm.at[pl.ds(0, L)], bufs[t], sems[t]).wait()

        def consume(b, t):
            drain(t)
            buf = bufs[t]
            zero = jnp.zeros((16,), jnp.float32)

            def make_rbody(h):
                def rbody(rr, acc):
                    w = bits_v[pl.ds((2 * b + h) * WPC + (rr >> 5), 16)][0]
                    off = (lax.shift_right_logical(w, rr & 31) & 1) * D
                    r = h * HALF + rr
                    return tuple(acc[d] + buf[r, pl.ds(off + d * 16, 16)]
                                 for d in range(NVREG))
                return rbody

            acc = lax.fori_loop(0, HALF, make_rbody(0), (zero,) * NVREG,
                                unroll=8)
            acc = lax.fori_loop(0, HALF, make_rbody(1), acc, unroll=8)
            inv = jnp.float32(1.0 / L)
            for d in range(NVREG):
                out_v[b, pl.ds(d * 16, 16)] = acc[d] * inv

        issue(0, 0)

        def outer(i, carry):
            for t in range(2):
                b = 2 * i + t

                @pl.when(b + 1 < rpw)
                def _():
                    issue(b + 1, (t + 1) % 2)

                consume(b, t)
            return carry

        lax.fori_loop(0, rpw // 2, outer, 0)
        pltpu.sync_copy(out_v, out_hbm.at[pl.ds(base, rpw)])

    return pool(gtext, pbits, emb2)


def _mlp_body(x_ref, w1_ref, b1_ref, w2_ref, b2_ref, o_ref):
    x = x_ref[...]
    h = lax.dot_general(x, w1_ref[...], (((1,), (1,)), ((), ())),
                        preferred_element_type=jnp.float32)
    h = jnp.maximum(h + b1_ref[...], 0.0)
    o = lax.dot_general(h, w2_ref[...], (((1,), (1,)), ((), ())),
                        preferred_element_type=jnp.float32)
    o_ref[...] = o + b2_ref[...]


def _mlp_tc(pooled, W1, b1, W2, b2):
    blk = 512
    return pl.pallas_call(
        _mlp_body,
        grid=(B // blk,),
        in_specs=[
            pl.BlockSpec((blk, D), lambda i: (i, 0)),
            pl.BlockSpec((H, D), lambda i: (0, 0)),
            pl.BlockSpec((1, H), lambda i: (0, 0)),
            pl.BlockSpec((O, H), lambda i: (0, 0)),
            pl.BlockSpec((1, O), lambda i: (0, 0)),
        ],
        out_specs=pl.BlockSpec((blk, O), lambda i: (i, 0)),
        out_shape=jax.ShapeDtypeStruct((B, O), jnp.float32),
    )(pooled, W1, b1.reshape(1, H), W2, b2.reshape(1, O))


def kernel(text, emb, W1, b1, W2, b2):
    ti = text.astype(jnp.int32)
    gtext = lax.shift_right_logical(ti, 1).reshape(2 * B, HALF)
    par = (ti & 1).reshape(2 * B, HALF)
    parp = jnp.pad(par, ((0, 0), (0, 28))).reshape(2 * B, WPC, 32)
    shifts = jnp.arange(32, dtype=jnp.int32)[None, None, :]
    pbits = (parp << shifts).sum(axis=2, dtype=jnp.int32).reshape(-1)
    emb2 = emb.reshape(emb.shape[0] // 2, 2 * D)
    pooled = _pool_sc(gtext, pbits, emb2)
    return _mlp_tc(pooled, W1, b1, W2, b2)


# speedup vs baseline: 1.3974x; 1.2330x over previous
import functools

import jax
import jax.numpy as jnp
from jax import lax
from jax.experimental import pallas as pl
from jax.experimental.pallas import tpu as pltpu
from jax.experimental.pallas import tpu_sc as plsc

B = 4096
L = 200
D = 64
H = 128
O = 10
HALF = L // 2
NVREG = D // 16
WPC = 4


def _pool_sc(gtext, pbits, emb2):
    info = plsc.get_sparse_core_info()
    ncores = info.num_cores
    nw = ncores * info.num_subcores
    rpw = B // nw
    nchunk = 2 * rpw
    mesh = plsc.VectorSubcoreMesh(core_axis_name="c", subcore_axis_name="s")

    @functools.partial(
        pl.kernel,
        out_type=jax.ShapeDtypeStruct((B, D), jnp.float32),
        mesh=mesh,
        scratch_types=[
            pltpu.VMEM((nchunk, HALF), jnp.int32),
            pltpu.VMEM((nchunk * WPC + 16,), jnp.int32),
            pltpu.VMEM((L, 2 * D), jnp.float32),
            pltpu.VMEM((L, 2 * D), jnp.float32),
            pltpu.VMEM((rpw, D), jnp.float32),
            pltpu.SemaphoreType.DMA,
            pltpu.SemaphoreType.DMA,
        ],
    )
    def pool(gtext_hbm, pbits_hbm, emb_hbm, out_hbm,
             gidx_v, bits_v, rows0, rows1, out_v, sem0, sem1):
        wid = lax.axis_index("s") * ncores + lax.axis_index("c")
        base = wid * rpw
        pltpu.sync_copy(gtext_hbm.at[pl.ds(2 * base, nchunk)], gidx_v)
        pltpu.sync_copy(pbits_hbm.at[pl.ds(2 * base * WPC, nchunk * WPC)],
                        bits_v.at[pl.ds(0, nchunk * WPC)])
        bufs = (rows0, rows1)
        sems = (sem0, sem1)

        def issue(b, t):
            pltpu.async_copy(emb_hbm.at[gidx_v.at[2 * b]],
                             bufs[t].at[pl.ds(0, HALF)], sems[t])
            pltpu.async_copy(emb_hbm.at[gidx_v.at[2 * b + 1]],
                             bufs[t].at[pl.ds(HALF, HALF)], sems[t])

        def drain(t):
            pltpu.make_async_copy(emb_hbm.at[pl.ds(0, L)], bufs[t], sems[t]).wait()

        def consume(b, t):
            drain(t)
            buf = bufs[t]
            zero = jnp.zeros((16,), jnp.float32)

            def make_rbody(h):
                def rbody(rr, acc):
                    w = bits_v[pl.ds((2 * b + h) * WPC + (rr >> 5), 16)][0]
                    off = (lax.shift_right_logical(w, rr & 31) & 1) * D
                    r = h * HALF + rr
                    return tuple(acc[d] + buf[r, pl.ds(off + d * 16, 16)]
                                 for d in range(NVREG))
                return rbody

            acc = lax.fori_loop(0, HALF, make_rbody(0), (zero,) * NVREG,
                                unroll=8)
            acc = lax.fori_loop(0, HALF, make_rbody(1), acc, unroll=8)
            inv = jnp.float32(1.0 / L)
            for d in range(NVREG):
                out_v[b, pl.ds(d * 16, 16)] = acc[d] * inv

        issue(0, 0)

        def outer(i, carry):
            for t in range(2):
                b = 2 * i + t

                @pl.when(b + 1 < rpw)
                def _():
                    issue(b + 1, (t + 1) % 2)

                consume(b, t)
            return carry

        lax.fori_loop(0, rpw // 2, outer, 0)
        pltpu.sync_copy(out_v, out_hbm.at[pl.ds(base, rpw)])

    return pool(gtext, pbits, emb2)


def _tr_body(x_ref, o_ref):
    x3 = x_ref[...].T.reshape(2048, 2, D)
    o_ref[:, 0:D] = x3[:, 0, :]
    o_ref[:, D:2 * D] = x3[:, 1, :]


def _transpose_tc(embT):
    V = embT.shape[1]
    K = 4096
    grid = (V + K - 1) // K
    return pl.pallas_call(
        _tr_body,
        grid=(grid,),
        in_specs=[pl.BlockSpec((D, K), lambda i: (0, i))],
        out_specs=pl.BlockSpec((K // 2, 2 * D), lambda i: (i, 0)),
        out_shape=jax.ShapeDtypeStruct((V // 2, 2 * D), jnp.float32),
    )(embT)


def _mlp_body(x_ref, w1_ref, b1_ref, w2_ref, b2_ref, o_ref):
    x = x_ref[...]
    h = lax.dot_general(x, w1_ref[...], (((1,), (1,)), ((), ())),
                        preferred_element_type=jnp.float32)
    h = jnp.maximum(h + b1_ref[...], 0.0)
    o = lax.dot_general(h, w2_ref[...], (((1,), (1,)), ((), ())),
                        preferred_element_type=jnp.float32)
    o_ref[...] = o + b2_ref[...]


def _mlp_tc(pooled, W1, b1, W2, b2):
    blk = 512
    return pl.pallas_call(
        _mlp_body,
        grid=(B // blk,),
        in_specs=[
            pl.BlockSpec((blk, D), lambda i: (i, 0)),
            pl.BlockSpec((H, D), lambda i: (0, 0)),
            pl.BlockSpec((1, H), lambda i: (0, 0)),
            pl.BlockSpec((O, H), lambda i: (0, 0)),
            pl.BlockSpec((1, O), lambda i: (0, 0)),
        ],
        out_specs=pl.BlockSpec((blk, O), lambda i: (i, 0)),
        out_shape=jax.ShapeDtypeStruct((B, O), jnp.float32),
    )(pooled, W1, b1.reshape(1, H), W2, b2.reshape(1, O))


def kernel(text, emb, W1, b1, W2, b2):
    ti = text.astype(jnp.int32)
    gtext = lax.shift_right_logical(ti, 1).reshape(2 * B, HALF)
    par = (ti & 1).reshape(2 * B, HALF)
    parp = jnp.pad(par, ((0, 0), (0, 28))).reshape(2 * B, WPC, 32)
    shifts = jnp.arange(32, dtype=jnp.int32)[None, None, :]
    pbits = (parp << shifts).sum(axis=2, dtype=jnp.int32).reshape(-1)
    emb2 = _transpose_tc(emb.T)
    pooled = _pool_sc(gtext, pbits, emb2)
    return _mlp_tc(pooled, W1, b1, W2, b2)


# speedup vs baseline: 1.5169x; 1.0855x over previous
import functools

import jax
import jax.numpy as jnp
from jax import lax
from jax.experimental import pallas as pl
from jax.experimental.pallas import tpu as pltpu
from jax.experimental.pallas import tpu_sc as plsc

B = 4096
L = 200
D = 64
H = 128
O = 10
HALF = L // 2
NVREG = D // 16
WPC = 4


def _pool_sc(text2, emb2):
    info = plsc.get_sparse_core_info()
    ncores = info.num_cores
    nw = ncores * info.num_subcores
    rpw = B // nw
    nchunk = 2 * rpw
    mesh = plsc.VectorSubcoreMesh(core_axis_name="c", subcore_axis_name="s")

    @functools.partial(
        pl.kernel,
        out_type=jax.ShapeDtypeStruct((B, D), jnp.float32),
        mesh=mesh,
        scratch_types=[
            pltpu.VMEM((nchunk, HALF), jnp.int32),
            pltpu.VMEM((L, 2 * D), jnp.float32),
            pltpu.VMEM((L, 2 * D), jnp.float32),
            pltpu.VMEM((rpw, D), jnp.float32),
            pltpu.SemaphoreType.DMA,
            pltpu.SemaphoreType.DMA,
        ],
    )
    def pool(text_hbm, emb_hbm, out_hbm,
             gidx_v, rows0, rows1, out_v, sem0, sem1):
        wid = lax.axis_index("s") * ncores + lax.axis_index("c")
        base = wid * rpw
        pltpu.sync_copy(text_hbm.at[pl.ds(2 * base, nchunk)], gidx_v)
        bufs = (rows0, rows1)
        sems = (sem0, sem1)

        def issue(b, t):
            pltpu.async_copy(emb_hbm.at[gidx_v.at[2 * b]],
                             bufs[t].at[pl.ds(0, HALF)], sems[t])
            pltpu.async_copy(emb_hbm.at[gidx_v.at[2 * b + 1]],
                             bufs[t].at[pl.ds(HALF, HALF)], sems[t])

        def drain(t):
            pltpu.make_async_copy(emb_hbm.at[pl.ds(0, L)], bufs[t], sems[t]).wait()

        def consume(b, t):
            drain(t)
            buf = bufs[t]
            zero = jnp.zeros((16,), jnp.float32)

            def rbody(r, acc):
                return tuple(acc[d] + buf[r, pl.ds(d * 16, 16)]
                             for d in range(NVREG))

            acc = lax.fori_loop(0, L, rbody, (zero,) * NVREG, unroll=8)
            inv = jnp.float32(1.0 / L)
            for d in range(NVREG):
                out_v[b, pl.ds(d * 16, 16)] = acc[d] * inv

        issue(0, 0)

        def outer(i, carry):
            for t in range(2):
                b = 2 * i + t

                @pl.when(b + 1 < rpw)
                def _():
                    issue(b + 1, (t + 1) % 2)

                consume(b, t)
            return carry

        lax.fori_loop(0, rpw // 2, outer, 0)
        pltpu.sync_copy(out_v, out_hbm.at[pl.ds(base, rpw)])

    return pool(text2, emb2)


def _tr_body(x_ref, o_ref):
    xt = x_ref[...].T
    o_ref[:, 0:D] = xt
    o_ref[:, D:2 * D] = xt


def _transpose_tc(embT):
    V = embT.shape[1]
    K = 4096
    grid = (V + K - 1) // K
    return pl.pallas_call(
        _tr_body,
        grid=(grid,),
        in_specs=[pl.BlockSpec((D, K), lambda i: (0, i))],
        out_specs=pl.BlockSpec((K, 2 * D), lambda i: (i, 0)),
        out_shape=jax.ShapeDtypeStruct((V, 2 * D), jnp.float32),
    )(embT)


def _mlp_body(x_ref, w1_ref, b1_ref, w2_ref, b2_ref, o_ref):
    x = x_ref[...]
    h = lax.dot_general(x, w1_ref[...], (((1,), (1,)), ((), ())),
                        preferred_element_type=jnp.float32)
    h = jnp.maximum(h + b1_ref[...], 0.0)
    o = lax.dot_general(h, w2_ref[...], (((1,), (1,)), ((), ())),
                        preferred_element_type=jnp.float32)
    o_ref[...] = o + b2_ref[...]


def _mlp_tc(pooled, W1, b1, W2, b2):
    blk = 512
    return pl.pallas_call(
        _mlp_body,
        grid=(B // blk,),
        in_specs=[
            pl.BlockSpec((blk, D), lambda i: (i, 0)),
            pl.BlockSpec((H, D), lambda i: (0, 0)),
            pl.BlockSpec((1, H), lambda i: (0, 0)),
            pl.BlockSpec((O, H), lambda i: (0, 0)),
            pl.BlockSpec((1, O), lambda i: (0, 0)),
        ],
        out_specs=pl.BlockSpec((blk, O), lambda i: (i, 0)),
        out_shape=jax.ShapeDtypeStruct((B, O), jnp.float32),
    )(pooled, W1, b1.reshape(1, H), W2, b2.reshape(1, O))


def kernel(text, emb, W1, b1, W2, b2):
    text2 = text.astype(jnp.int32).reshape(2 * B, HALF)
    emb2 = _transpose_tc(emb.T)
    pooled = _pool_sc(text2, emb2)
    return _mlp_tc(pooled, W1, b1, W2, b2)
